# Initial kernel scaffold; baseline (speedup 1.0000x reference)
import jax
import jax.numpy as jnp
from jax.experimental import pallas as pl


def kernel(feats, edge_index, edge_attr, W_red, b_red, Wq0, Wk0, Wv0, Wo0, bq0, bk0, bv0, bo0, W10, b10, W20, b20, ln1w0, ln1b0, ln2w0, ln2b0, Wq1, Wk1, Wv1, Wo1, bq1, bk1, bv1, bo1, W11, b11, W21, b21, ln1w1, ln1b1, ln2w1, ln2b1):
    raise NotImplementedError("write your pallas kernel here")


# speedup vs baseline: 10.5763x; 10.5763x over previous
import functools
import jax
import jax.numpy as jnp
import numpy as np
from jax import lax
from jax.experimental import pallas as pl
from jax.experimental.pallas import tpu as pltpu

_N = 10000
_E = 320000
_DIN = 512
_D = 128
_H = 8
_HD = 16
_DH = 512

_RB = 1000
_EB = 4000

_NEG = -3.0e38


def _sel_mat():
    s = np.zeros((_D, _H), np.float32)
    for h in range(_H):
        s[h * _HD:(h + 1) * _HD, h] = 1.0
    return jnp.asarray(s)


def _pre_body(feats, wred, bred, wq, bq, wk, bk, wv, bv, x_o, q_o, k_o, v_o):
    x = jnp.dot(feats[...], wred[...], preferred_element_type=jnp.float32) + bred[...]
    x_o[...] = x
    q_o[...] = jnp.dot(x, wq[...], preferred_element_type=jnp.float32) + bq[...]
    k_o[...] = jnp.dot(x, wk[...], preferred_element_type=jnp.float32) + bk[...]
    v_o[...] = jnp.dot(x, wv[...], preferred_element_type=jnp.float32) + bv[...]


def _pre_call(feats, wred, bred, wq, bq, wk, bk, wv, bv):
    nsteps = _N // _RB
    full = lambda shape: pl.BlockSpec(shape, lambda i: (0, 0))
    row = lambda width: pl.BlockSpec((_RB, width), lambda i: (i, 0))
    return pl.pallas_call(
        _pre_body,
        grid=(nsteps,),
        in_specs=[row(_DIN), full((_DIN, _D)), full((1, _D)),
                  full((_D, _D)), full((1, _D)), full((_D, _D)), full((1, _D)),
                  full((_D, _D)), full((1, _D))],
        out_specs=[row(_D), row(_D), row(_D), row(_D)],
        out_shape=[jax.ShapeDtypeStruct((_N, _D), jnp.float32)] * 4,
    )(feats, wred, bred, wq, bq, wk, bk, wv, bv)


def _qkv_body(x, wq, bq, wk, bk, wv, bv, q_o, k_o, v_o):
    x = x[...]
    q_o[...] = jnp.dot(x, wq[...], preferred_element_type=jnp.float32) + bq[...]
    k_o[...] = jnp.dot(x, wk[...], preferred_element_type=jnp.float32) + bk[...]
    v_o[...] = jnp.dot(x, wv[...], preferred_element_type=jnp.float32) + bv[...]


def _qkv_call(x, wq, bq, wk, bk, wv, bv):
    nsteps = _N // _RB
    full = lambda shape: pl.BlockSpec(shape, lambda i: (0, 0))
    row = lambda width: pl.BlockSpec((_RB, width), lambda i: (i, 0))
    return pl.pallas_call(
        _qkv_body,
        grid=(nsteps,),
        in_specs=[row(_D), full((_D, _D)), full((1, _D)), full((_D, _D)),
                  full((1, _D)), full((_D, _D)), full((1, _D))],
        out_specs=[row(_D), row(_D), row(_D)],
        out_shape=[jax.ShapeDtypeStruct((_N, _D), jnp.float32)] * 3,
    )(x, wq, bq, wk, bk, wv, bv)


def _score_body(qc, kr, sel, s_o, m_o):
    i = pl.program_id(0)
    s = jnp.dot(qc[...] * kr[...], sel[...],
                preferred_element_type=jnp.float32) * (1.0 / 4.0)
    s_o[...] = s
    mb = jnp.max(s, axis=0, keepdims=True)

    @pl.when(i == 0)
    def _():
        m_o[...] = mb

    @pl.when(i > 0)
    def _():
        m_o[...] = jnp.maximum(m_o[...], mb)


def _score_call(qc, kr, sel):
    nsteps = _E // _EB
    return pl.pallas_call(
        _score_body,
        grid=(nsteps,),
        in_specs=[pl.BlockSpec((_EB, _D), lambda i: (i, 0)),
                  pl.BlockSpec((_EB, _D), lambda i: (i, 0)),
                  pl.BlockSpec((_D, _H), lambda i: (0, 0))],
        out_specs=[pl.BlockSpec((_EB, _H), lambda i: (i, 0)),
                   pl.BlockSpec((1, _H), lambda i: (0, 0))],
        out_shape=[jax.ShapeDtypeStruct((_E, _H), jnp.float32),
                   jax.ShapeDtypeStruct((1, _H), jnp.float32)],
    )(qc, kr, sel)


def _exmsg_body(s, m, vr, selt, ex_o, msg_o):
    ex = jnp.exp(s[...] - m[...])
    ex_o[...] = ex
    msg_o[...] = jnp.dot(ex, selt[...], preferred_element_type=jnp.float32) * vr[...]


def _exmsg_call(s, m, vr, selt):
    nsteps = _E // _EB
    return pl.pallas_call(
        _exmsg_body,
        grid=(nsteps,),
        in_specs=[pl.BlockSpec((_EB, _H), lambda i: (i, 0)),
                  pl.BlockSpec((1, _H), lambda i: (0, 0)),
                  pl.BlockSpec((_EB, _D), lambda i: (i, 0)),
                  pl.BlockSpec((_H, _D), lambda i: (0, 0))],
        out_specs=[pl.BlockSpec((_EB, _H), lambda i: (i, 0)),
                   pl.BlockSpec((_EB, _D), lambda i: (i, 0))],
        out_shape=[jax.ShapeDtypeStruct((_E, _H), jnp.float32),
                   jax.ShapeDtypeStruct((_E, _D), jnp.float32)],
    )(s, m, vr, selt)


def _ln(x, w, b):
    mu = jnp.mean(x, axis=-1, keepdims=True)
    xc = x - mu
    var = jnp.mean(xc * xc, axis=-1, keepdims=True)
    return xc * jax.lax.rsqrt(var + 1e-5) * w + b


def _post_body(x, agg0, agg1, den0, den1, selt, wo, bo, w1, b1, w2, b2,
               l1w, l1b, l2w, l2b, x_o):
    den = jnp.dot(den0[...] + den1[...], selt[...],
                  preferred_element_type=jnp.float32)
    agg = (agg0[...] + agg1[...]) / (den + 1e-16)
    f2 = jnp.dot(agg, wo[...], preferred_element_type=jnp.float32) + bo[...]
    y = _ln(x[...] + f2, l1w[...], l1b[...])
    z = jnp.dot(jnp.maximum(jnp.dot(y, w1[...], preferred_element_type=jnp.float32)
                            + b1[...], 0.0),
                w2[...], preferred_element_type=jnp.float32) + b2[...]
    x_o[...] = _ln(y + z, l2w[...], l2b[...])


def _post_call(x, agg0, agg1, den0, den1, selt, wo, bo, w1, b1, w2, b2,
               l1w, l1b, l2w, l2b):
    nsteps = _N // _RB
    full = lambda shape: pl.BlockSpec(shape, lambda i: (0, 0))
    row = lambda width: pl.BlockSpec((_RB, width), lambda i: (i, 0))
    return pl.pallas_call(
        _post_body,
        grid=(nsteps,),
        in_specs=[row(_D), row(_D), row(_D), row(_H), row(_H),
                  full((_H, _D)), full((_D, _D)), full((1, _D)),
                  full((_D, _DH)), full((1, _DH)), full((_DH, _D)), full((1, _D)),
                  full((1, _D)), full((1, _D)), full((1, _D)), full((1, _D))],
        out_specs=[row(_D)],
        out_shape=[jax.ShapeDtypeStruct((_N, _D), jnp.float32)],
    )(x, agg0, agg1, den0, den1, selt, wo, bo, w1, b1, w2, b2,
      l1w, l1b, l2w, l2b)[0]


def _gather_rows(table, idx):
    return jnp.take(table, idx, axis=0)


def _scatter_partials(ex, msg, c):
    den = jax.ops.segment_sum(ex, c, num_segments=_N)
    agg = jax.ops.segment_sum(msg, c, num_segments=_N)
    z8 = jnp.zeros_like(den)
    zd = jnp.zeros_like(agg)
    return (den, z8), (agg, zd)


def kernel(feats, edge_index, edge_attr, W_red, b_red,
           Wq0, Wk0, Wv0, Wo0, bq0, bk0, bv0, bo0,
           W10, b10, W20, b20, ln1w0, ln1b0, ln2w0, ln2b0,
           Wq1, Wk1, Wv1, Wo1, bq1, bk1, bv1, bo1,
           W11, b11, W21, b21, ln1w1, ln1b1, ln2w1, ln2b1):
    sel = _sel_mat()
    selt = sel.T
    r = edge_index[:, 0]
    c = edge_index[:, 1]
    v2 = lambda a: a.reshape(1, -1)

    x, q, k, v = _pre_call(feats, W_red, v2(b_red),
                           Wq0, v2(bq0), Wk0, v2(bk0), Wv0, v2(bv0))

    layers = [
        (Wq0, bq0, Wk0, bk0, Wv0, bv0, Wo0, bo0, W10, b10, W20, b20,
         ln1w0, ln1b0, ln2w0, ln2b0),
        (Wq1, bq1, Wk1, bk1, Wv1, bv1, Wo1, bo1, W11, b11, W21, b21,
         ln1w1, ln1b1, ln2w1, ln2b1),
    ]
    for li, (wq, bq, wk, bk, wv, bv, wo, bo, w1, b1, w2, b2,
             l1w, l1b, l2w, l2b) in enumerate(layers):
        if li > 0:
            q, k, v = _qkv_call(x, wq, v2(bq), wk, v2(bk), wv, v2(bv))
        qc = _gather_rows(q, c)
        kr = _gather_rows(k, r)
        vr = _gather_rows(v, r)
        s, m = _score_call(qc, kr, sel)
        ex, msg = _exmsg_call(s, m, vr, selt)
        (den0, den1), (agg0, agg1) = _scatter_partials(ex, msg, c)
        x = _post_call(x, agg0, agg1, den0, den1, selt, wo, v2(bo),
                       w1, v2(b1), w2, v2(b2), v2(l1w), v2(l1b),
                       v2(l2w), v2(l2b))
    return x


# speedup vs baseline: 34.9055x; 3.3004x over previous
import functools
import jax
import jax.numpy as jnp
import numpy as np
from jax import lax
from jax.experimental import pallas as pl
from jax.experimental.pallas import tpu as pltpu
from jax.experimental.pallas import tpu_sc as plsc

_N = 10000
_E = 320000
_DIN = 512
_D = 128
_H = 8
_HD = 16
_DH = 512

_RB = 1000
_EB = 4000

_NEG = -3.0e38


def _sel_mat():
    s = np.zeros((_D, _H), np.float32)
    for h in range(_H):
        s[h * _HD:(h + 1) * _HD, h] = 1.0
    return jnp.asarray(s)


def _pre_body(feats, wred, bred, wq, bq, wk, bk, wv, bv, x_o, q_o, k_o, v_o):
    x = jnp.dot(feats[...], wred[...], preferred_element_type=jnp.float32) + bred[...]
    x_o[...] = x
    q_o[...] = jnp.dot(x, wq[...], preferred_element_type=jnp.float32) + bq[...]
    k_o[...] = jnp.dot(x, wk[...], preferred_element_type=jnp.float32) + bk[...]
    v_o[...] = jnp.dot(x, wv[...], preferred_element_type=jnp.float32) + bv[...]


def _pre_call(feats, wred, bred, wq, bq, wk, bk, wv, bv):
    nsteps = _N // _RB
    full = lambda shape: pl.BlockSpec(shape, lambda i: (0, 0))
    row = lambda width: pl.BlockSpec((_RB, width), lambda i: (i, 0))
    return pl.pallas_call(
        _pre_body,
        grid=(nsteps,),
        in_specs=[row(_DIN), full((_DIN, _D)), full((1, _D)),
                  full((_D, _D)), full((1, _D)), full((_D, _D)), full((1, _D)),
                  full((_D, _D)), full((1, _D))],
        out_specs=[row(_D), row(_D), row(_D), row(_D)],
        out_shape=[jax.ShapeDtypeStruct((_N, _D), jnp.float32)] * 4,
    )(feats, wred, bred, wq, bq, wk, bk, wv, bv)


def _qkv_body(x, wq, bq, wk, bk, wv, bv, q_o, k_o, v_o):
    x = x[...]
    q_o[...] = jnp.dot(x, wq[...], preferred_element_type=jnp.float32) + bq[...]
    k_o[...] = jnp.dot(x, wk[...], preferred_element_type=jnp.float32) + bk[...]
    v_o[...] = jnp.dot(x, wv[...], preferred_element_type=jnp.float32) + bv[...]


def _qkv_call(x, wq, bq, wk, bk, wv, bv):
    nsteps = _N // _RB
    full = lambda shape: pl.BlockSpec(shape, lambda i: (0, 0))
    row = lambda width: pl.BlockSpec((_RB, width), lambda i: (i, 0))
    return pl.pallas_call(
        _qkv_body,
        grid=(nsteps,),
        in_specs=[row(_D), full((_D, _D)), full((1, _D)), full((_D, _D)),
                  full((1, _D)), full((_D, _D)), full((1, _D))],
        out_specs=[row(_D), row(_D), row(_D)],
        out_shape=[jax.ShapeDtypeStruct((_N, _D), jnp.float32)] * 3,
    )(x, wq, bq, wk, bk, wv, bv)


def _score_body(qc, kr, sel, s_o, m_o):
    i = pl.program_id(0)
    s = jnp.dot(qc[...] * kr[...], sel[...],
                preferred_element_type=jnp.float32) * (1.0 / 4.0)
    s_o[...] = s
    mb = jnp.max(s, axis=0, keepdims=True)

    @pl.when(i == 0)
    def _():
        m_o[...] = mb

    @pl.when(i > 0)
    def _():
        m_o[...] = jnp.maximum(m_o[...], mb)


def _score_call(qc, kr, sel):
    nsteps = _E // _EB
    return pl.pallas_call(
        _score_body,
        grid=(nsteps,),
        in_specs=[pl.BlockSpec((_EB, _D), lambda i: (i, 0)),
                  pl.BlockSpec((_EB, _D), lambda i: (i, 0)),
                  pl.BlockSpec((_D, _H), lambda i: (0, 0))],
        out_specs=[pl.BlockSpec((_EB, _H), lambda i: (i, 0)),
                   pl.BlockSpec((1, _H), lambda i: (0, 0))],
        out_shape=[jax.ShapeDtypeStruct((_E, _H), jnp.float32),
                   jax.ShapeDtypeStruct((1, _H), jnp.float32)],
    )(qc, kr, sel)


def _exmsg_body(s, m, vr, selt, exrep_o, msg_o):
    ex = jnp.exp(s[...] - m[...])
    exrep = jnp.dot(ex, selt[...], preferred_element_type=jnp.float32)
    exrep_o[...] = exrep
    msg_o[...] = exrep * vr[...]


def _exmsg_call(s, m, vr, selt):
    nsteps = _E // _EB
    return pl.pallas_call(
        _exmsg_body,
        grid=(nsteps,),
        in_specs=[pl.BlockSpec((_EB, _H), lambda i: (i, 0)),
                  pl.BlockSpec((1, _H), lambda i: (0, 0)),
                  pl.BlockSpec((_EB, _D), lambda i: (i, 0)),
                  pl.BlockSpec((_H, _D), lambda i: (0, 0))],
        out_specs=[pl.BlockSpec((_EB, _D), lambda i: (i, 0)),
                   pl.BlockSpec((_EB, _D), lambda i: (i, 0))],
        out_shape=[jax.ShapeDtypeStruct((_E, _D), jnp.float32),
                   jax.ShapeDtypeStruct((_E, _D), jnp.float32)],
    )(s, m, vr, selt)


def _ln(x, w, b):
    mu = jnp.mean(x, axis=-1, keepdims=True)
    xc = x - mu
    var = jnp.mean(xc * xc, axis=-1, keepdims=True)
    return xc * jax.lax.rsqrt(var + 1e-5) * w + b


def _post_body(x, agg_i, den_i, wo, bo, w1, b1, w2, b2,
               l1w, l1b, l2w, l2b, x_o):
    agg = agg_i[...] / (den_i[...] + 1e-16)
    f2 = jnp.dot(agg, wo[...], preferred_element_type=jnp.float32) + bo[...]
    y = _ln(x[...] + f2, l1w[...], l1b[...])
    z = jnp.dot(jnp.maximum(jnp.dot(y, w1[...], preferred_element_type=jnp.float32)
                            + b1[...], 0.0),
                w2[...], preferred_element_type=jnp.float32) + b2[...]
    x_o[...] = _ln(y + z, l2w[...], l2b[...])


def _post_call(x, agg, den, wo, bo, w1, b1, w2, b2,
               l1w, l1b, l2w, l2b):
    nsteps = _N // _RB
    full = lambda shape: pl.BlockSpec(shape, lambda i: (0, 0))
    row = lambda width: pl.BlockSpec((_RB, width), lambda i: (i, 0))
    return pl.pallas_call(
        _post_body,
        grid=(nsteps,),
        in_specs=[row(_D), row(_D), row(_D),
                  full((_D, _D)), full((1, _D)),
                  full((_D, _DH)), full((1, _DH)), full((_DH, _D)), full((1, _D)),
                  full((1, _D)), full((1, _D)), full((1, _D)), full((1, _D))],
        out_specs=[row(_D)],
        out_shape=[jax.ShapeDtypeStruct((_N, _D), jnp.float32)],
    )(x, agg, den, wo, bo, w1, b1, w2, b2,
      l1w, l1b, l2w, l2b)[0]


_NC = 2
_NS = 16
_NW = _NC * _NS
_CH = 80
_EW = _E // _NW
_NCH = _EW // _CH
_NP = 10240
_NR = _NP // _NS


def _gather3_build():
    mesh = plsc.VectorSubcoreMesh(core_axis_name="c", subcore_axis_name="s")

    @functools.partial(
        pl.kernel, mesh=mesh,
        out_type=[jax.ShapeDtypeStruct((_E, _D), jnp.float32)] * 3,
        scratch_types=[
            pltpu.VMEM((_NCH, _CH), jnp.int32),
            pltpu.VMEM((_NCH, _CH), jnp.int32),
            pltpu.VMEM((_CH, _D), jnp.float32),
            pltpu.VMEM((_CH, _D), jnp.float32),
            pltpu.VMEM((_CH, _D), jnp.float32),
            pltpu.SemaphoreType.DMA,
            pltpu.SemaphoreType.DMA,
            pltpu.SemaphoreType.DMA,
        ],
    )
    def gat(cidx_h, ridx_h, q_h, k_h, v_h, qc_o, kr_o, vr_o,
            cv, rv, qb, kb, vb, sq, sk, sv):
        wid = lax.axis_index("s") * _NC + lax.axis_index("c")
        pltpu.sync_copy(cidx_h.at[wid], cv)
        pltpu.sync_copy(ridx_h.at[wid], rv)
        base = wid * _EW

        def body(j, carry):
            cq = pltpu.async_copy(q_h.at[cv.at[j]], qb, sq)
            ck = pltpu.async_copy(k_h.at[rv.at[j]], kb, sk)
            cv2 = pltpu.async_copy(v_h.at[rv.at[j]], vb, sv)
            cq.wait()
            ck.wait()
            cv2.wait()
            off = base + j * _CH
            pltpu.sync_copy(qb, qc_o.at[pl.ds(off, _CH)])
            pltpu.sync_copy(kb, kr_o.at[pl.ds(off, _CH)])
            pltpu.sync_copy(vb, vr_o.at[pl.ds(off, _CH)])
            return carry

        lax.fori_loop(0, _NCH, body, 0)

    return gat


_ES = _E // _NS
_NCHS = _ES // _CH


def _scatter2_build():
    mesh = plsc.VectorSubcoreMesh(core_axis_name="c", subcore_axis_name="s")

    @functools.partial(
        pl.kernel, mesh=mesh,
        out_type=[jax.ShapeDtypeStruct((_NP, _D), jnp.float32),
                  jax.ShapeDtypeStruct((_NP, _D), jnp.float32)],
        scratch_types=[
            pltpu.VMEM((_NCHS, _CH), jnp.int32),
            pltpu.VMEM((_CH, _D), jnp.float32),
            pltpu.VMEM_SHARED((_NP, _D), jnp.float32),
        ],
    )
    def sca(cidx_h, msg_h, exrep_h, zero_h, agg_o, den_o,
            cv, buf, acc_sh):
        cid = lax.axis_index("c")
        sid = lax.axis_index("s")
        rb = sid * _NR
        pltpu.sync_copy(zero_h.at[pl.ds(rb, _NR)], acc_sh.at[pl.ds(rb, _NR)])
        pltpu.sync_copy(cidx_h.at[sid], cv)
        plsc.subcore_barrier()
        base = sid * _ES

        @pl.when(cid == 0)
        def _():
            def body(j, carry):
                off = base + j * _CH
                pltpu.sync_copy(msg_h.at[pl.ds(off, _CH)], buf)
                pltpu.sync_copy(buf, acc_sh.at[cv.at[j]], add=True)
                return carry
            lax.fori_loop(0, _NCHS, body, 0)

        @pl.when(cid == 1)
        def _():
            def body(j, carry):
                off = base + j * _CH
                pltpu.sync_copy(exrep_h.at[pl.ds(off, _CH)], buf)
                pltpu.sync_copy(buf, acc_sh.at[cv.at[j]], add=True)
                return carry
            lax.fori_loop(0, _NCHS, body, 0)

        plsc.subcore_barrier()

        @pl.when(cid == 0)
        def _():
            pltpu.sync_copy(acc_sh.at[pl.ds(rb, _NR)], agg_o.at[pl.ds(rb, _NR)])

        @pl.when(cid == 1)
        def _():
            pltpu.sync_copy(acc_sh.at[pl.ds(rb, _NR)], den_o.at[pl.ds(rb, _NR)])

    return sca


_g3 = _gather3_build()
_s2 = _scatter2_build()


def kernel(feats, edge_index, edge_attr, W_red, b_red,
           Wq0, Wk0, Wv0, Wo0, bq0, bk0, bv0, bo0,
           W10, b10, W20, b20, ln1w0, ln1b0, ln2w0, ln2b0,
           Wq1, Wk1, Wv1, Wo1, bq1, bk1, bv1, bo1,
           W11, b11, W21, b21, ln1w1, ln1b1, ln2w1, ln2b1):
    sel = _sel_mat()
    selt = sel.T
    r3 = edge_index[:, 0].reshape(_NW, _NCH, _CH)
    c3 = edge_index[:, 1].reshape(_NW, _NCH, _CH)
    cs3 = edge_index[:, 1].reshape(_NS, _NCHS, _CH)
    zacc = jnp.zeros((_NP, _D), jnp.float32)
    v2 = lambda a: a.reshape(1, -1)

    x, q, k, v = _pre_call(feats, W_red, v2(b_red),
                           Wq0, v2(bq0), Wk0, v2(bk0), Wv0, v2(bv0))

    layers = [
        (Wq0, bq0, Wk0, bk0, Wv0, bv0, Wo0, bo0, W10, b10, W20, b20,
         ln1w0, ln1b0, ln2w0, ln2b0),
        (Wq1, bq1, Wk1, bk1, Wv1, bv1, Wo1, bo1, W11, b11, W21, b21,
         ln1w1, ln1b1, ln2w1, ln2b1),
    ]
    for li, (wq, bq, wk, bk, wv, bv, wo, bo, w1, b1, w2, b2,
             l1w, l1b, l2w, l2b) in enumerate(layers):
        if li > 0:
            q, k, v = _qkv_call(x, wq, v2(bq), wk, v2(bk), wv, v2(bv))
        qc, kr, vr = _g3(c3, r3, q, k, v)
        s, m = _score_call(qc, kr, sel)
        exrep, msg = _exmsg_call(s, m, vr, selt)
        agg, den = _s2(cs3, msg, exrep, zacc)
        x = _post_call(x, agg[:_N], den[:_N], wo, v2(bo),
                       w1, v2(b1), w2, v2(b2), v2(l1w), v2(l1b),
                       v2(l2w), v2(l2b))
    return x
